# baseline (speedup 1.0000x reference)
import functools

import jax
import jax.numpy as jnp
from jax import lax
from jax.experimental import pallas as pl
from jax.experimental.pallas import tpu as pltpu
from jax.experimental.pallas import tpu_sc as plsc

N_SEGMENT = 1000
D_MODEL = 1024
N_TOKENS = 4 * 8192

_NC = 2
_NS = 16
_NW = _NC * _NS
_TOK_PER_W = N_TOKENS // _NW
_CHUNK = 32
_NCHUNK = _TOK_PER_W // _CHUNK

_mesh = plsc.VectorSubcoreMesh(core_axis_name="c", subcore_axis_name="s")


@functools.partial(
    pl.kernel,
    mesh=_mesh,
    out_type=jax.ShapeDtypeStruct((N_TOKENS, D_MODEL), jnp.float32),
    scratch_types=[
        pltpu.VMEM((_TOK_PER_W,), jnp.int32),
        pltpu.VMEM((2, _CHUNK, D_MODEL), jnp.float32),
        pltpu.SemaphoreType.DMA,
        pltpu.SemaphoreType.DMA,
        pltpu.SemaphoreType.DMA,
    ],
)
def _emb_lookup(x_hbm, w_hbm, out_hbm, idx_v, rows_v, gsem, ssem0, ssem1):
    sid = lax.axis_index("s")
    wid = sid * _NC + lax.axis_index("c")
    base = wid * _TOK_PER_W
    ssem = (ssem0, ssem1)

    pltpu.sync_copy(x_hbm.at[pl.ds(base, _TOK_PER_W)], idx_v)

    def gather(g, s):
        return pltpu.async_copy(
            w_hbm.at[idx_v.at[pl.ds(g * _CHUNK, _CHUNK)]],
            rows_v.at[s], gsem,
        )

    def scatter(g, s):
        return pltpu.async_copy(
            rows_v.at[s], out_hbm.at[pl.ds(base + g * _CHUNK, _CHUNK)],
            ssem[s],
        )

    pending = [None, None]
    gather(0, 0).wait()
    for g in range(1, _NCHUNK):
        s = g & 1
        pending[1 - s] = scatter(g - 1, 1 - s)
        if pending[s] is not None:
            pending[s].wait()
            pending[s] = None
        gather(g, s).wait()
    last = _NCHUNK - 1
    pending[last & 1] = scatter(last, last & 1)
    for p in pending:
        if p is not None:
            p.wait()


def kernel(x, weight):
    out = _emb_lookup(x.reshape(N_TOKENS), weight)
    return out.reshape(x.shape[0], x.shape[1], D_MODEL)


# speedup vs baseline: 1.0110x; 1.0110x over previous
import functools

import jax
import jax.numpy as jnp
from jax import lax
from jax.experimental import pallas as pl
from jax.experimental.pallas import tpu as pltpu
from jax.experimental.pallas import tpu_sc as plsc

N_SEGMENT = 1000
D_MODEL = 1024
N_TOKENS = 4 * 8192

_NC = 2
_NS = 16
_NW = _NC * _NS
_TOK_PER_W = N_TOKENS // _NW
_CHUNK = 32
_NBUF = 3
_NCHUNK = _TOK_PER_W // _CHUNK

_mesh = plsc.VectorSubcoreMesh(core_axis_name="c", subcore_axis_name="s")


@functools.partial(
    pl.kernel,
    mesh=_mesh,
    out_type=jax.ShapeDtypeStruct((N_TOKENS, D_MODEL), jnp.float32),
    scratch_types=[
        pltpu.VMEM((_TOK_PER_W,), jnp.int32),
        pltpu.VMEM((_NBUF, _CHUNK, D_MODEL), jnp.float32),
    ]
    + [pltpu.SemaphoreType.DMA] * (2 * _NBUF),
)
def _emb_lookup(x_hbm, w_hbm, out_hbm, idx_v, rows_v, *sems):
    sid = lax.axis_index("s")
    wid = sid * _NC + lax.axis_index("c")
    base = wid * _TOK_PER_W
    gsem = sems[:_NBUF]
    ssem = sems[_NBUF:]

    pltpu.sync_copy(x_hbm.at[pl.ds(base, _TOK_PER_W)], idx_v)

    def gather(g, s):
        return pltpu.async_copy(
            w_hbm.at[idx_v.at[pl.ds(g * _CHUNK, _CHUNK)]],
            rows_v.at[s], gsem[s],
        )

    def scatter(g, s):
        return pltpu.async_copy(
            rows_v.at[s], out_hbm.at[pl.ds(base + g * _CHUNK, _CHUNK)],
            ssem[s],
        )

    gpend = [None] * _NBUF
    spend = [None] * _NBUF
    ahead = _NBUF - 1
    for g in range(min(ahead, _NCHUNK)):
        gpend[g % _NBUF] = gather(g, g % _NBUF)
    for g in range(_NCHUNK):
        s = g % _NBUF
        nxt = g + ahead
        if nxt < _NCHUNK:
            ns = nxt % _NBUF
            if spend[ns] is not None:
                spend[ns].wait()
                spend[ns] = None
            gpend[ns] = gather(nxt, ns)
        gpend[s].wait()
        gpend[s] = None
        spend[s] = scatter(g, s)
    for p in spend:
        if p is not None:
            p.wait()


def kernel(x, weight):
    out = _emb_lookup(x.reshape(N_TOKENS), weight)
    return out.reshape(x.shape[0], x.shape[1], D_MODEL)
